# Initial kernel scaffold; baseline (speedup 1.0000x reference)
import jax
import jax.numpy as jnp
from jax.experimental import pallas as pl


def kernel(x, Wr, br, W1, b1, W2, b2):
    raise NotImplementedError("write your pallas kernel here")


# speedup vs baseline: 3.7609x; 3.7609x over previous
import functools

import jax
import jax.numpy as jnp
from jax import lax
from jax.experimental import pallas as pl
from jax.experimental.pallas import tpu as pltpu
from jax.experimental.pallas import tpu_sc as plsc

S = 2048
D = 768
E = 8
H = 3072
NA = 2 * S
T = 256
NB = 24
NR = NB * T

NC, NS = 2, 16
NW = NC * NS


def _logits_body(wr_ref, x_ref, br_ref, lg_ref):
    lg_ref[...] = lax.dot_general(
        wr_ref[...], x_ref[...], (((0,), (1,)), ((), ())),
        preferred_element_type=jnp.float32) + br_ref[...]


def _logits(x_flat, Wr, br8):
    blk = 128
    return pl.pallas_call(
        _logits_body,
        grid=(S // blk,),
        in_specs=[
            pl.BlockSpec((D, E), lambda b: (0, 0)),
            pl.BlockSpec((blk, D), lambda b: (b, 0)),
            pl.BlockSpec((E, 1), lambda b: (0, 0)),
        ],
        out_specs=pl.BlockSpec((E, blk), lambda b: (0, b)),
        out_shape=jax.ShapeDtypeStruct((E, S), jnp.float32),
    )(Wr, x_flat, br8)


def _routing_body(lg_ref, w_ref, rows_ref, be_ref, ranks_ref, oh_ref):
    lg = lg_ref[...]
    it = lax.broadcasted_iota(jnp.int32, (E, S), 0)
    m1 = jnp.max(lg, axis=0, keepdims=True)
    idx0 = jnp.min(jnp.where(lg == m1, it, E), axis=0, keepdims=True)
    l2 = jnp.where(it == idx0, -jnp.inf, lg)
    m2 = jnp.max(l2, axis=0, keepdims=True)
    idx1 = jnp.min(jnp.where(l2 == m2, it, E), axis=0, keepdims=True)
    s1 = jnp.exp(m2 - m1)
    den = 1.0 + s1
    w_ref[0:1, :] = 1.0 / den
    w_ref[1:2, :] = s1 / den

    oh_ref[:, 0:S] = jnp.where(it == idx0, 1.0, 0.0)
    oh_ref[:, S:NA] = jnp.where(it == idx1, 1.0, 0.0)

    ci = lax.broadcasted_iota(jnp.int32, (128, 128), 0)
    cj = lax.broadcasted_iota(jnp.int32, (128, 128), 1)
    tril = jnp.where(ci < cj, 1.0, 0.0)

    def step(i, carry):
        chunk = oh_ref[:, pl.ds(i * 128, 128)]
        r = lax.dot_general(chunk, tril, (((1,), (0,)), ((), ())),
                            preferred_element_type=jnp.float32) + carry
        ranks_ref[:, pl.ds(i * 128, 128)] = r
        return carry + jnp.sum(chunk, axis=1, keepdims=True)

    counts = lax.fori_loop(0, NA // 128, step, jnp.zeros((E, 1), jnp.float32))

    tf = jnp.float32(T)
    padded = jnp.floor((counts + (tf - 1.0)) / tf) * tf
    ei = lax.broadcasted_iota(jnp.int32, (E, E), 0)
    ej = lax.broadcasted_iota(jnp.int32, (E, E), 1)
    mlow = jnp.where(ej < ei, 1.0, 0.0)
    base = lax.dot_general(mlow, padded, (((1,), (0,)), ((), ())),
                           preferred_element_type=jnp.float32)

    rows = jnp.sum(oh_ref[...] * (ranks_ref[...] + base), axis=0,
                   keepdims=True)
    rows_ref[...] = rows.astype(jnp.int32)

    bt = lax.broadcasted_iota(jnp.int32, (E, NB), 1).astype(jnp.float32) * tf
    cnt = jnp.sum(jnp.where(bt >= base, 1.0, 0.0), axis=0, keepdims=True)
    total = jnp.sum(padded, axis=0, keepdims=True)
    used = (lax.broadcasted_iota(jnp.int32, (1, NB), 1).astype(jnp.float32)
            * tf < total)
    be_ref[...] = jnp.where(used, cnt - 1.0, -1.0).astype(jnp.int32)


def _routing(lg):
    return pl.pallas_call(
        _routing_body,
        out_shape=(
            jax.ShapeDtypeStruct((2, S), jnp.float32),
            jax.ShapeDtypeStruct((1, NA), jnp.int32),
            jax.ShapeDtypeStruct((1, NB), jnp.int32),
        ),
        scratch_shapes=[pltpu.VMEM((E, NA), jnp.float32),
                        pltpu.VMEM((E, NA), jnp.float32)],
    )(lg)


def _dispatch_body(x_hbm, rows_hbm, xs_hbm, idxa, idxb, buf, sema, semb):
    wid = lax.axis_index("s") * NC + lax.axis_index("c")
    tpw = S // NW
    base = wid * tpw
    pltpu.sync_copy(rows_hbm.at[pl.ds(base, tpw)], idxa)
    pltpu.sync_copy(rows_hbm.at[pl.ds(S + base, tpw)], idxb)
    pltpu.sync_copy(x_hbm.at[pl.ds(base, tpw)], buf)
    ca = pltpu.async_copy(buf, xs_hbm.at[idxa], sema)
    cb = pltpu.async_copy(buf, xs_hbm.at[idxb], semb)
    ca.wait()
    cb.wait()


def _dispatch(x_flat, rows):
    tpw = S // NW
    f = pl.kernel(
        _dispatch_body,
        out_type=jax.ShapeDtypeStruct((NR, D), jnp.float32),
        mesh=plsc.VectorSubcoreMesh(core_axis_name="c", subcore_axis_name="s"),
        scratch_types=[
            pltpu.VMEM((tpw,), jnp.int32),
            pltpu.VMEM((tpw,), jnp.int32),
            pltpu.VMEM((tpw, D), jnp.float32),
            pltpu.SemaphoreType.DMA,
            pltpu.SemaphoreType.DMA,
        ],
    )
    return f(x_flat, rows)


def _mlp_body(be_ref, xs_ref, w1_ref, b1_ref, w2_ref, b2_ref, out_ref):
    be = be_ref[pl.program_id(0)]

    @pl.when(be >= 0)
    def _():
        h = jnp.dot(xs_ref[...], w1_ref[0], preferred_element_type=jnp.float32)
        h = h + b1_ref[0]
        h = 0.5 * h * (1.0 + lax.erf(h * 0.7071067811865476))
        out_ref[...] = jnp.dot(h, w2_ref[0],
                               preferred_element_type=jnp.float32) + b2_ref[0]


def _mlp(be, xs, W1, b1r, W2, b2r):
    def wsel(b, be_ref):
        return (jnp.maximum(be_ref[b], 0), 0, 0)

    grid_spec = pltpu.PrefetchScalarGridSpec(
        num_scalar_prefetch=1,
        grid=(NB,),
        in_specs=[
            pl.BlockSpec((T, D), lambda b, be_ref: (b, 0)),
            pl.BlockSpec((1, D, H), wsel),
            pl.BlockSpec((1, 1, H), wsel),
            pl.BlockSpec((1, H, D), wsel),
            pl.BlockSpec((1, 1, D), wsel),
        ],
        out_specs=pl.BlockSpec((T, D), lambda b, be_ref: (b, 0)),
    )
    return pl.pallas_call(
        _mlp_body,
        grid_spec=grid_spec,
        out_shape=jax.ShapeDtypeStruct((NR, D), jnp.float32),
    )(be, xs, W1, b1r, W2, b2r)


def _gather_body(ys_hbm, rows_hbm, g_hbm, idx, buf, sem):
    wid = lax.axis_index("s") * NC + lax.axis_index("c")
    apw = NA // NW
    base = wid * apw
    pltpu.sync_copy(rows_hbm.at[pl.ds(base, apw)], idx)
    pltpu.async_copy(ys_hbm.at[idx], buf, sem).wait()
    pltpu.sync_copy(buf, g_hbm.at[pl.ds(base, apw)])


def _gather(ys, rows):
    apw = NA // NW
    f = pl.kernel(
        _gather_body,
        out_type=jax.ShapeDtypeStruct((NA, D), jnp.float32),
        mesh=plsc.VectorSubcoreMesh(core_axis_name="c", subcore_axis_name="s"),
        scratch_types=[
            pltpu.VMEM((apw,), jnp.int32),
            pltpu.VMEM((apw, D), jnp.float32),
            pltpu.SemaphoreType.DMA,
        ],
    )
    return f(ys, rows)


def _combine_body(ga_ref, gb_ref, wt_ref, out_ref):
    out_ref[...] = (ga_ref[...] * wt_ref[:, 0:1] +
                    gb_ref[...] * wt_ref[:, 1:2])


def _combine(g, wt):
    blk = 256
    return pl.pallas_call(
        _combine_body,
        grid=(S // blk,),
        in_specs=[
            pl.BlockSpec((blk, D), lambda b: (b, 0)),
            pl.BlockSpec((blk, D), lambda b: (b + S // blk, 0)),
            pl.BlockSpec((blk, 2), lambda b: (b, 0)),
        ],
        out_specs=pl.BlockSpec((blk, D), lambda b: (b, 0)),
        out_shape=jax.ShapeDtypeStruct((S, D), jnp.float32),
    )(g, g, wt)


def kernel(x, Wr, br, W1, b1, W2, b2):
    b, s, d = x.shape
    x_flat = x.reshape(S, D)
    lg = _logits(x_flat, Wr, br.reshape(E, 1))
    w2t, rows1, be1 = _routing(lg)
    wt = w2t.T
    rows = rows1.reshape(NA)
    be = be1.reshape(NB)
    xs = _dispatch(x_flat, rows)
    ys = _mlp(be, xs, W1, b1.reshape(E, 1, H), W2, b2.reshape(E, 1, D))
    g = _gather(ys, rows)
    out = _combine(g, wt)
    return out.reshape(b, s, d)
